# Initial kernel scaffold; baseline (speedup 1.0000x reference)
import jax
import jax.numpy as jnp
from jax.experimental import pallas as pl


def kernel(x, W_lin, b_lin, select_weight, select_mean_diff, imp_slope, imp_center, lora_update, lora_degen, bias_p, debias_p):
    raise NotImplementedError("write your pallas kernel here")


# speedup vs baseline: 4.9372x; 4.9372x over previous
import functools

import jax
import jax.numpy as jnp
from jax.experimental import pallas as pl

_N = 8
_S = 4
_H = 8
_ETA = 1.0


def _glo_kernel(x_ref, w_ref, b_ref, mean_ref, wsel_ref, slope_ref,
                center_ref, u_ref, g_ref, bias_ref, debias_ref, out_ref):
    f32 = jnp.float32
    x_blk = x_ref[...]
    x_lin = jax.lax.dot_general(
        x_blk, w_ref[...], (((1,), (1,)), ((), ())),
        preferred_element_type=f32) + b_ref[...]

    r2 = jnp.sum(x_lin * x_lin, axis=1, keepdims=True)
    mean = mean_ref[...]
    xm = jax.lax.dot_general(
        x_lin, mean, (((1,), (1,)), ((), ())),
        preferred_element_type=f32)
    wsel = wsel_ref[...]
    proj = jax.lax.dot_general(
        x_lin, wsel, (((1,), (0,)), ((), ())),
        preferred_element_type=f32)
    mw_full = jax.lax.dot_general(
        mean, wsel, (((1,), (0,)), ((), ())),
        preferred_element_type=f32)
    cols_s = jax.lax.broadcasted_iota(jnp.int32, (_N, _N * _S), 1) // _S
    rows_s = jax.lax.broadcasted_iota(jnp.int32, (_N, _N * _S), 0)
    mw_diag = jnp.sum(jnp.where(cols_s == rows_s, mw_full, 0.0),
                      axis=0, keepdims=True)
    proj = proj - mw_diag

    best = None
    idx = None
    for n in range(_N):
        m2_n = jnp.sum(mean[n, :] * mean[n, :])
        d2_n = r2 - 2.0 * xm[:, n:n + 1] + m2_n
        p = proj[:, n * _S:(n + 1) * _S]
        cont = jnp.sum(p * p, axis=1, keepdims=True) / d2_n
        score = slope_ref[0, n] * (cont - center_ref[0, n])
        sel_n = jax.nn.sigmoid(score)
        if n == 0:
            best = sel_n
            idx = jnp.zeros_like(sel_n, dtype=jnp.int32)
        else:
            upd = sel_n > best
            best = jnp.where(upd, sel_n, best)
            idx = jnp.where(upd, n, idx)

    u_all = jax.lax.dot_general(
        x_lin, u_ref[...], (((1,), (0,)), ((), ())),
        preferred_element_type=f32)
    c_full = jax.lax.dot_general(
        debias_ref[...], u_ref[...], (((1,), (0,)), ((), ())),
        preferred_element_type=f32)
    cols_h = jax.lax.broadcasted_iota(jnp.int32, (_N, _N * _H), 1) // _H
    rows_h = jax.lax.broadcasted_iota(jnp.int32, (_N, _N * _H), 0)
    c_diag = jnp.sum(jnp.where(cols_h == rows_h, c_full, 0.0),
                     axis=0, keepdims=True)

    tb = x_blk.shape[0]
    lbl_h = jax.lax.broadcasted_iota(jnp.int32, (tb, _N * _H), 1) // _H
    oh_h = (lbl_h == idx).astype(f32)
    w_masked = oh_h * (u_all - c_diag)
    degen_up = jax.lax.dot_general(
        w_masked, g_ref[...], (((1,), (0,)), ((), ())),
        preferred_element_type=f32)

    lbl_n = jax.lax.broadcasted_iota(jnp.int32, (tb, _N), 1)
    oh_n = (lbl_n == idx).astype(f32)
    bias_sel = jax.lax.dot_general(
        oh_n, bias_ref[...], (((1,), (0,)), ((), ())),
        preferred_element_type=f32)

    mod_x_bias = _ETA * (bias_sel + degen_up)
    out_ref[...] = (1.0 - best) * x_lin + best * mod_x_bias


@functools.partial(jax.jit, static_argnames=())
def kernel(x, W_lin, b_lin, select_weight, select_mean_diff, imp_slope,
           imp_center, lora_update, lora_degen, bias_p, debias_p):
    B, T, D = x.shape
    N, _, S = select_weight.shape
    H = lora_update.shape[2]
    x2 = x.reshape(B * T, D)
    wsel = jnp.transpose(select_weight, (1, 0, 2)).reshape(D, N * S)
    u2 = jnp.transpose(lora_update, (1, 0, 2)).reshape(D, N * H)
    g2 = jnp.transpose(lora_degen, (0, 2, 1)).reshape(N * H, D)
    slope = imp_slope.reshape(1, N)
    center = imp_center.reshape(1, N)
    b2 = b_lin.reshape(1, D)

    TB = 512
    grid = ((B * T) // TB,)
    const = lambda shape: pl.BlockSpec(shape, lambda i: (0, 0))
    out = pl.pallas_call(
        _glo_kernel,
        grid=grid,
        in_specs=[
            pl.BlockSpec((TB, D), lambda i: (i, 0)),
            const((D, D)),
            const((1, D)),
            const((N, D)),
            const((D, N * S)),
            const((1, N)),
            const((1, N)),
            const((D, N * H)),
            const((N * H, D)),
            const((N, D)),
            const((N, D)),
        ],
        out_specs=pl.BlockSpec((TB, D), lambda i: (i, 0)),
        out_shape=jax.ShapeDtypeStruct((B * T, D), jnp.float32),
    )(x2, W_lin, b2, select_mean_diff, wsel, slope, center, u2, g2,
      bias_p, debias_p)
    return out.reshape(B, T, D)


# speedup vs baseline: 6.1425x; 1.2441x over previous
import jax
import jax.numpy as jnp
from jax.experimental import pallas as pl
from jax.experimental.pallas import tpu as pltpu

_N = 8
_S = 4
_H = 8
_ETA = 1.0


def _glo_kernel(x_ref, w_ref, b_ref, wcat_ref, mw_ref, m2_ref, slope_ref,
                center_ref, gcat_ref, out_ref):
    f32 = jnp.float32
    x_blk = x_ref[...]
    x_lin = jax.lax.dot_general(
        x_blk, w_ref[...], (((1,), (1,)), ((), ())),
        preferred_element_type=f32) + b_ref[...]

    aux = jax.lax.dot_general(
        x_lin, wcat_ref[...], (((1,), (0,)), ((), ())),
        preferred_element_type=f32)
    u_all = aux[:, :_N * _H]
    proj = aux[:, _N * _H:_N * _H + _N * _S] - mw_ref[...]
    xm = aux[:, _N * _H + _N * _S:]

    r2 = jnp.sum(x_lin * x_lin, axis=1, keepdims=True)
    d2 = r2 - 2.0 * xm + m2_ref[...]
    q = proj * proj
    smat = (jax.lax.broadcasted_iota(jnp.int32, (_N * _S, _N), 0) // _S ==
            jax.lax.broadcasted_iota(jnp.int32, (_N * _S, _N), 1)).astype(f32)
    qsum = jax.lax.dot_general(
        q, smat, (((1,), (0,)), ((), ())),
        preferred_element_type=f32)
    score = slope_ref[...] * (qsum / d2 - center_ref[...])

    rowmax = jnp.max(score, axis=1, keepdims=True)
    tb = x_blk.shape[0]
    iota_n = jax.lax.broadcasted_iota(jnp.int32, (tb, _N), 1)
    idx = jnp.min(jnp.where(score == rowmax, iota_n, _N),
                  axis=1, keepdims=True)
    ss = jax.nn.sigmoid(rowmax)

    nh = _N * _H
    vals = jnp.concatenate(
        [u_all, jnp.ones((tb, _N), dtype=f32)], axis=1)
    lbl = jax.lax.broadcasted_iota(jnp.int32, (tb, nh + _N), 1)
    lbl = jnp.where(lbl < nh, lbl // _H, lbl - nh)
    masked = jnp.where(lbl == idx, vals, 0.0)
    upd = jax.lax.dot_general(
        masked, gcat_ref[...], (((1,), (0,)), ((), ())),
        preferred_element_type=f32)

    out_ref[...] = x_lin + ss * (_ETA * upd - x_lin)


def kernel(x, W_lin, b_lin, select_weight, select_mean_diff, imp_slope,
           imp_center, lora_update, lora_degen, bias_p, debias_p):
    B, T, D = x.shape
    N, _, S = select_weight.shape
    H = lora_update.shape[2]
    x2 = x.reshape(B * T, D)
    b2 = b_lin.reshape(1, D)

    wsel = jnp.transpose(select_weight, (1, 0, 2)).reshape(D, N * S)
    u2 = jnp.transpose(lora_update, (1, 0, 2)).reshape(D, N * H)
    wcat = jnp.concatenate([u2, wsel, select_mean_diff.T], axis=1)
    mw = jnp.einsum('nd,nds->ns', select_mean_diff,
                    select_weight).reshape(1, N * S)
    m2 = jnp.sum(select_mean_diff * select_mean_diff, axis=1).reshape(1, N)
    c = jnp.einsum('nd,ndh->nh', debias_p, lora_update)
    cb = jnp.einsum('nh,ndh->nd', c, lora_degen)
    bias_eff = bias_p - cb
    g2 = jnp.transpose(lora_degen, (0, 2, 1)).reshape(N * H, D)
    gcat = jnp.concatenate([g2, bias_eff], axis=0)
    slope = imp_slope.reshape(1, N)
    center = imp_center.reshape(1, N)

    TB = 512
    grid = ((B * T) // TB,)
    const = lambda shape: pl.BlockSpec(shape, lambda i: (0, 0))
    out = pl.pallas_call(
        _glo_kernel,
        grid=grid,
        in_specs=[
            pl.BlockSpec((TB, D), lambda i: (i, 0)),
            const((D, D)),
            const((1, D)),
            const((D, N * (H + S + 1))),
            const((1, N * S)),
            const((1, N)),
            const((1, N)),
            const((1, N)),
            const((N * (H + 1), D)),
        ],
        out_specs=pl.BlockSpec((TB, D), lambda i: (i, 0)),
        out_shape=jax.ShapeDtypeStruct((B * T, D), jnp.float32),
        compiler_params=pltpu.CompilerParams(
            dimension_semantics=("parallel",)),
    )(x2, W_lin, b2, wcat, mw, m2, slope, center, gcat)
    return out.reshape(B, T, D)
